# baseline (speedup 1.0000x reference)
import functools
import math

import jax
import jax.numpy as jnp
from jax import lax
from jax.experimental import pallas as pl
from jax.experimental.pallas import tpu as pltpu, tpu_sc as plsc

NUM_RBF = 18
_MAGIC = 0x5F3759DF


def _rsqrt(x, iters=1):
    i = plsc.bitcast(x, jnp.int32)
    y = plsc.bitcast(jnp.int32(_MAGIC) - (i >> 1), jnp.float32)
    for _ in range(iters):
        y = y * (1.5 - 0.5 * x * y * y)
    return y


def _sqrt(x):
    return x * _rsqrt(jnp.maximum(x, 1e-35))


def _bf16r(x):
    u = plsc.bitcast(x, jnp.int32)
    r = (u + jnp.int32(0x7FFF) + ((u >> 16) & 1)) & jnp.int32(-65536)
    return plsc.bitcast(r, jnp.float32)


def _normalize3(v0, v1, v2, iters=1):
    ss = v0 * v0 + v1 * v1 + v2 * v2
    inv = _rsqrt(jnp.maximum(ss, 1e-30), iters)
    z = jnp.where(ss > 0.0, inv, 0.0)
    return v0 * z, v1 * z, v2 * z


def _cross(a, b):
    return (
        a[1] * b[2] - a[2] * b[1],
        a[2] * b[0] - a[0] * b[2],
        a[0] * b[1] - a[1] * b[0],
    )


def _sc_geo(co_i32, dst3, idx3, *, B, L, K):
    NT = 32
    TPB = NT // B
    KQ = K // TPB
    KH = KQ // 2
    BL = 128
    NBLK = L // BL
    LQ = L // TPB
    CH = NUM_RBF + 7
    IW = BL * K

    mesh = plsc.VectorSubcoreMesh(core_axis_name="c", subcore_axis_name="s",
                                  num_cores=2, num_subcores=16)

    @functools.partial(
        pl.kernel,
        out_type=[
            jax.ShapeDtypeStruct((3 * B * L,), jnp.float32),
            jax.ShapeDtypeStruct((B * CH, K, L), jnp.float32),
        ],
        mesh=mesh,
        compiler_params=pltpu.CompilerParams(needs_layout_passes=False),
        scratch_types=[
            pltpu.VMEM((12 * L,), jnp.float32),
            pltpu.VMEM((CH, KH, BL), jnp.float32),
            pltpu.VMEM((CH, KH, BL), jnp.float32),
            pltpu.VMEM((KH, BL), jnp.int32),
            pltpu.VMEM((KH, BL), jnp.int32),
            pltpu.VMEM((KH, BL), jnp.float32),
            pltpu.VMEM((KH, BL), jnp.float32),
            pltpu.VMEM((3 * L,), jnp.float32),
            pltpu.VMEM((3 * L,), jnp.float32),
            pltpu.SemaphoreType.DMA,
            pltpu.SemaphoreType.DMA,
            pltpu.SemaphoreType.DMA,
            pltpu.SemaphoreType.DMA,
            pltpu.SemaphoreType.DMA,
            pltpu.SemaphoreType.DMA,
        ],
    )
    def body(co_hbm, dst_hbm, idx_hbm, node_hbm, edge_hbm,
             tab, stA, stB, ibA, ibB, dbA, dbB, cab, adbuf,
             si0, si1, sd0, sd1, sA, sB):
        cid = lax.axis_index("c")
        sid = lax.axis_index("s")
        wid = sid * 2 + cid
        b = wid // TPB
        kq = wid % TPB

        for c in range(3):
            pltpu.sync_copy(co_hbm.at[b, c, 1], cab.at[pl.ds(c * L, L)])

        iota = lax.iota(jnp.int32, 16)
        eps = 1e-6

        @pl.loop(0, L // 16)
        def _node(g):
            lane = g * 16 + iota
            ms = [jnp.clip(lane + o, 0, L - 1) for o in (-1, 0, 1, 2)]
            xs = []
            for m in ms:
                xs.append([plsc.load_gather(cab, [jnp.int32(c * L) + m])
                           for c in range(3)])
            u2 = _normalize3(*[xs[1][c] - xs[0][c] for c in range(3)], iters=2)
            u1 = _normalize3(*[xs[2][c] - xs[1][c] for c in range(3)], iters=2)
            u0 = _normalize3(*[xs[3][c] - xs[2][c] for c in range(3)], iters=2)
            n2 = _normalize3(*_cross(u2, u1), iters=2)
            n1 = _normalize3(*_cross(u1, u0), iters=2)
            cosA = -(u1[0] * u0[0] + u1[1] * u0[1] + u1[2] * u0[2])
            cosA = jnp.clip(cosA, -1 + eps, 1 - eps)
            cosD = n2[0] * n1[0] + n2[1] * n1[1] + n2[2] * n1[2]
            cosD = jnp.clip(cosD, -1 + eps, 1 - eps)
            sinA = _sqrt(1.0 - cosA * cosA)
            sgn = jnp.sign(u2[0] * n1[0] + u2[1] * n1[1] + u2[2] * n1[2])
            sinD = _sqrt(1.0 - cosD * cosD) * sgn
            o1 = _normalize3(u2[0] - u1[0], u2[1] - u1[1], u2[2] - u1[2], iters=2)
            o3 = _cross(o1, n2)
            validf = jnp.where((lane >= 1) & (lane <= L - 3), 1.0, 0.0)
            orows = [o1[0], o1[1], o1[2], n2[0], n2[1], n2[2], o3[0], o3[1], o3[2]]
            for c in range(9):
                tab[pl.ds(c * L + g * 16, 16)] = _bf16r(orows[c] * validf)
            for c in range(3):
                tab[pl.ds((9 + c) * L + g * 16, 16)] = xs[1][c]
            ad = [cosA, sinA * cosD, sinA * sinD]
            for c in range(3):
                adbuf[pl.ds(c * L + g * 16, 16)] = ad[c] * validf

        k0A = kq * KQ
        pltpu.async_copy(idx_hbm.at[b, pl.ds(k0A, KH), pl.ds(0, BL)], ibA, si0)
        pltpu.async_copy(dst_hbm.at[b, pl.ds(k0A, KH), pl.ds(0, BL)], dbA, sd0)

        for c in range(3):
            pltpu.sync_copy(adbuf.at[pl.ds(c * L + kq * LQ, LQ)],
                            node_hbm.at[pl.ds(c * (B * L) + b * L + kq * LQ, LQ)])

        delta = 20.0 / (NUM_RBF - 1)
        inv_sig = NUM_RBF / 20.0
        tk = 2.0 * delta * inv_sig * inv_sig
        cms = [math.exp(-((m * delta * inv_sig) ** 2)) for m in range(NUM_RBF)]
        stages = (stA, stB)
        ssems = (sA, sB)
        ibs = (ibA, ibB)
        dbs = (dbA, dbB)
        isems = (si0, si1)
        dsems = (sd0, sd1)

        def do_block(blk):
            for kh in range(2):
                st = stages[kh]
                sem = ssems[kh]
                ib = ibs[kh]
                db = dbs[kh]

                nblk = blk + kh
                nk0 = kq * KQ + (kh ^ 1) * KH

                @pl.when(nblk < NBLK)
                def _():
                    pltpu.async_copy(
                        idx_hbm.at[b, pl.ds(nk0, KH), pl.ds(nblk * BL, BL)],
                        ibs[kh ^ 1], isems[kh ^ 1])
                    pltpu.async_copy(
                        dst_hbm.at[b, pl.ds(nk0, KH), pl.ds(nblk * BL, BL)],
                        dbs[kh ^ 1], dsems[kh ^ 1])

                pltpu.make_async_copy(
                    idx_hbm.at[b, pl.ds(0, KH), pl.ds(0, BL)], ib, isems[kh]).wait()
                pltpu.make_async_copy(
                    dst_hbm.at[b, pl.ds(0, KH), pl.ds(0, BL)], db, dsems[kh]).wait()

                @pl.when(blk >= 1)
                def _():
                    pltpu.make_async_copy(
                        st, edge_hbm.at[pl.ds(0, CH), pl.ds(0, KH), pl.ds(0, BL)],
                        sem).wait()

                @plsc.parallel_loop(0, BL // 16, unroll=2)
                def _lg(lg):
                    lloc = lg * 16 + iota
                    lvec = blk * BL + lloc
                    own = [plsc.load_gather(tab, [jnp.int32(c2 * L) + lvec])
                           for c2 in range(12)]
                    sl = pl.ds(lg * 16, 16)

                    for kk in range(KH):
                        idxv = ib[kk, sl]
                        Dv = db[kk, sl]
                        gj = [plsc.load_gather(tab, [jnp.int32(c2 * L) + idxv])
                              for c2 in range(12)]
                        z = Dv * inv_sig
                        e0 = jnp.exp(-(z * z))
                        tpow = jnp.exp(Dv * tk)
                        st[0, kk, sl] = e0
                        pw = e0
                        for m in range(1, NUM_RBF):
                            pw = pw * tpow
                            st[m, kk, sl] = pw * cms[m]
                        d = [_bf16r(gj[9 + c2] - own[9 + c2]) for c2 in range(3)]
                        t = [own[r * 3 + 0] * d[0] + own[r * 3 + 1] * d[1]
                             + own[r * 3 + 2] * d[2] for r in range(3)]
                        dU = _normalize3(*t)
                        for c2 in range(3):
                            st[NUM_RBF + c2, kk, sl] = dU[c2]
                        R = [[own[0 * 3 + a] * gj[0 * 3 + c2]
                              + own[1 * 3 + a] * gj[1 * 3 + c2]
                              + own[2 * 3 + a] * gj[2 * 3 + c2]
                              for c2 in range(3)] for a in range(3)]
                        tr0, tr1, tr2 = R[0][0], R[1][1], R[2][2]
                        a0 = jnp.abs(1.0 + tr0 - tr1 - tr2)
                        a1 = jnp.abs(1.0 - tr0 + tr1 - tr2)
                        a2 = jnp.abs(1.0 - tr0 - tr1 + tr2)
                        aw = jnp.maximum(1.0 + tr0 + tr1 + tr2, 0.0)
                        s0 = jnp.sign(R[2][1] - R[1][2])
                        s1 = jnp.sign(R[0][2] - R[2][0])
                        s2 = jnp.sign(R[1][0] - R[0][1])
                        qs = s0 * s0 * a0 + s1 * s1 * a1 + s2 * s2 * a2 + aw
                        invq = jnp.where(qs > 0.0,
                                         _rsqrt(jnp.maximum(qs, 1e-30)), 0.0)
                        st[NUM_RBF + 3, kk, sl] = s0 * _sqrt(a0) * invq
                        st[NUM_RBF + 4, kk, sl] = s1 * _sqrt(a1) * invq
                        st[NUM_RBF + 5, kk, sl] = s2 * _sqrt(a2) * invq
                        st[NUM_RBF + 6, kk, sl] = _sqrt(aw) * invq

                pltpu.async_copy(
                    st,
                    edge_hbm.at[pl.ds(b * CH, CH),
                                pl.ds(kq * KQ + kh * KH, KH),
                                pl.ds(blk * BL, BL)],
                    sem)

        @pl.loop(0, NBLK)
        def _blocks(blk):
            do_block(blk)

        pltpu.make_async_copy(
            stA, edge_hbm.at[pl.ds(0, CH), pl.ds(0, KH), pl.ds(0, BL)], sA).wait()
        pltpu.make_async_copy(
            stB, edge_hbm.at[pl.ds(0, CH), pl.ds(0, KH), pl.ds(0, BL)], sB).wait()

    return body(co_i32, dst3, idx3)


def kernel(coords, pairwise_dists, edge_ids, mask):
    B, L, K = pairwise_dists.shape
    CH = NUM_RBF + 7
    co4 = coords.transpose(0, 3, 2, 1)
    idx2 = edge_ids.astype(jnp.int32).transpose(0, 2, 1)
    dst2 = pairwise_dists.transpose(0, 2, 1)
    node_flat, edge3 = _sc_geo(co4, dst2, idx2, B=B, L=L, K=K)
    node = node_flat.reshape(3, B, L).transpose(1, 2, 0)
    edge = edge3.reshape(B, CH, K, L).transpose(0, 3, 2, 1)
    return node, edge
